# Initial kernel scaffold; baseline (speedup 1.0000x reference)
import jax
import jax.numpy as jnp
from jax.experimental import pallas as pl


def kernel(x, sequences, W0, b0, W1, b1):
    raise NotImplementedError("write your pallas kernel here")


# speedup vs baseline: 2.8988x; 2.8988x over previous
import jax
import jax.numpy as jnp
from jax.experimental import pallas as pl

RING_LO = 4
NSLOT = 128
B, S, E, H = 16, 2048, 256, 64
SCALE = H ** -0.5


def _body(x_ref, seq_ref, w0_ref, b0_ref, w1_ref, b1_ref, out_ref):
    xb = x_ref[0]
    seq = seq_ref[0]

    out0 = jnp.dot(xb, w0_ref[...], preferred_element_type=jnp.float32)
    out0 = out0 + b0_ref[...]
    out1 = jnp.dot(xb, w1_ref[...], preferred_element_type=jnp.float32)
    out1 = out1 + b1_ref[...]

    jv = jax.lax.broadcasted_iota(jnp.int32, (NSLOT, S), 0) + RING_LO
    sv = jax.lax.broadcasted_iota(jnp.int32, (NSLOT, S), 1)
    seqb = jnp.broadcast_to(seq, (NSLOT, S))
    cand = jnp.where(seqb == jv, sv, -1)
    sel = jnp.max(cand, axis=1, keepdims=True)
    onehot = ((cand == sel) & (sel >= 0)).astype(jnp.float32)

    slot_rows = jnp.dot(onehot, out1, preferred_element_type=jnp.float32)
    logits = jax.lax.dot_general(
        out0, slot_rows, (((1,), (1,)), ((), ())),
        preferred_element_type=jnp.float32)
    out_ref[0] = SCALE * logits[:, :100]


def kernel(x, sequences, W0, b0, W1, b1):
    seq3 = sequences.reshape(B, 1, S)
    b0r = b0.reshape(1, H)
    b1r = b1.reshape(1, H)
    return pl.pallas_call(
        _body,
        grid=(B,),
        in_specs=[
            pl.BlockSpec((1, S, E), lambda b: (b, 0, 0)),
            pl.BlockSpec((1, 1, S), lambda b: (b, 0, 0)),
            pl.BlockSpec((E, H), lambda b: (0, 0)),
            pl.BlockSpec((1, H), lambda b: (0, 0)),
            pl.BlockSpec((E, H), lambda b: (0, 0)),
            pl.BlockSpec((1, H), lambda b: (0, 0)),
        ],
        out_specs=pl.BlockSpec((1, S, 100), lambda b: (b, 0, 0)),
        out_shape=jax.ShapeDtypeStruct((B, S, 100), jnp.float32),
    )(x, seq3, W0, b0r, W1, b1r)
